# Initial kernel scaffold; baseline (speedup 1.0000x reference)
import jax
import jax.numpy as jnp
from jax.experimental import pallas as pl


def kernel(x0, x1, x2, up_index_0, boundary_index_1, down_index_1, up_index_1, boundary_index_2, down_index_2, batch0, batch1, batch2, W0, b0, W1, b1, W2, b2, W3, b3, W4, b4, W5, b5):
    raise NotImplementedError("write your pallas kernel here")


# speedup vs baseline: 15.6323x; 15.6323x over previous
import functools
import math

import jax
import jax.numpy as jnp
from jax import lax
from jax.experimental import pallas as pl
from jax.experimental.pallas import tpu as pltpu
from jax.experimental.pallas import tpu_sc as plsc

C = 128
NG = 64
N0, N1, N2 = 50000, 50000, 15000
NSLOT = 6
ROW = NSLOT * NG
L = 16
NC, NS = 2, 16
NW = NC * NS
CA = 512
CB = 1024

_f32 = jnp.float32
_i32 = jnp.int32

_SC_PARAMS = pltpu.CompilerParams(needs_layout_passes=False)
_MESH = plsc.VectorSubcoreMesh(core_axis_name="c", subcore_axis_name="s",
                               num_cores=NC, num_subcores=NS)


def _chunk_range(wid, total, chunk):
  nch = math.ceil(total / chunk)
  cpw = math.ceil(nch / NW)
  start = jnp.minimum(wid * cpw, nch)
  end = jnp.minimum(start + cpw, nch)
  return start, end, nch


def _wid():
  return lax.axis_index("s") * NC + lax.axis_index("c")


def _stage_a_body(x0r, x1r, x2r, wsumr,
                  t_up0, t_bd1, t_d1, t_u1, t_bd2, t_d2,
                  wtab, xrows, bufa, bufb, bufc, sem):
  wid = _wid()
  lanes = lax.iota(_i32, L)
  pltpu.sync_copy(wsumr, wtab)

  def do_job(xr, N, tables):
    start, end, _ = _chunk_range(wid, N, CA)

    def cbody(ci, carry):
      m0 = jnp.maximum(0, ci * CA - (N - CA))
      base = pl.multiple_of(ci * CA - m0, 8)
      pltpu.sync_copy(xr.at[pl.ds(base, CA), :], xrows)
      for t in range(CA // L):
        pos = lanes + (t * L)
        xv = [plsc.load_gather(xrows, [pos, jnp.full((L,), j, _i32)])
              for j in range(4)]
        for (out_ref, w_slot, buf) in tables:
          wsl = jnp.full((L,), w_slot, _i32)
          val = plsc.load_gather(wtab, [wsl, xv[0]])
          for j in range(1, 4):
            val = val + plsc.load_gather(wtab, [wsl, xv[j]])
          buf[pl.ds(t * L, L)] = val
      for (out_ref, w_slot, buf) in tables:
        pltpu.sync_copy(buf, out_ref.at[pl.ds(base, CA)])
      return carry

    lax.fori_loop(start, end, cbody, 0)

  do_job(x0r, N0, [(t_up0, 0, bufa)])
  do_job(x1r, N1, [(t_bd1, 1, bufa), (t_d1, 2, bufb), (t_u1, 3, bufc)])
  do_job(x2r, N2, [(t_bd2, 5, bufa), (t_d2, 4, bufb)])


_stage_a = functools.partial(
    pl.kernel,
    out_type=(
        jax.ShapeDtypeStruct((N0,), _f32),
        jax.ShapeDtypeStruct((N1,), _f32),
        jax.ShapeDtypeStruct((N1,), _f32),
        jax.ShapeDtypeStruct((N1,), _f32),
        jax.ShapeDtypeStruct((N2,), _f32),
        jax.ShapeDtypeStruct((N2,), _f32),
    ),
    mesh=_MESH,
    compiler_params=_SC_PARAMS,
    scratch_types=[
        pltpu.VMEM((NSLOT, C), _f32),
        pltpu.VMEM((CA, 4), _i32),
        pltpu.VMEM((CA,), _f32),
        pltpu.VMEM((CA,), _f32),
        pltpu.VMEM((CA,), _f32),
        pltpu.SemaphoreType.DMA,
    ],
)(_stage_a_body)


def _stage_b_body(s_up0, s_bd1, s_d1, s_u1, s_bd2, s_d2,
                  bat0, bat1, bat2,
                  up0v, up0g, bd1v, bd1g, d1v, d1g, u1v, u1g,
                  bd2v, bd2g, d2v, d2g, zr, outr,
                  acc, vidx, gidx, svals, gvals, sem):
  wid = _wid()
  lanes = lax.iota(_i32, L)
  lane_rows = lanes * ROW
  ones = jnp.ones((L,), _f32)

  pltpu.sync_copy(zr, acc)

  def do_edges(ev, eg, swr, batr, off, E):
    start, end, _ = _chunk_range(wid, E, CB)

    def cbody(ci, carry):
      m0 = jnp.maximum(0, ci * CB - (E - CB))
      base = pl.multiple_of(ci * CB - m0, 8)
      cp1 = pltpu.async_copy(ev.at[pl.ds(base, CB)], vidx, sem)
      cp2 = pltpu.async_copy(eg.at[pl.ds(base, CB)], gidx, sem)
      cp1.wait()
      cp2.wait()
      cps = []
      for j in range(CB // 128):
        sl = pl.ds(j * 128, 128)
        cps.append(pltpu.async_copy(swr.at[vidx.at[sl]], svals.at[sl], sem))
        cps.append(pltpu.async_copy(batr.at[gidx.at[sl]], gvals.at[sl], sem))
      for cp in cps:
        cp.wait()
      for t in range(CB // L):
        pos = lanes + (t * L)
        val = svals[pl.ds(t * L, L)]
        g = gvals[pl.ds(t * L, L)]
        mask = pos >= m0
        plsc.addupdate_scatter(acc, [lane_rows + (off + g)], val, mask=mask)
      return carry

    lax.fori_loop(start, end, cbody, 0)

  def do_counts(batr, N, off):
    start, end, _ = _chunk_range(wid, N, CB)

    def cbody(ci, carry):
      m0 = jnp.maximum(0, ci * CB - (N - CB))
      base = pl.multiple_of(ci * CB - m0, 8)
      pltpu.sync_copy(batr.at[pl.ds(base, CB)], gidx)
      for t in range(CB // L):
        pos = lanes + (t * L)
        g = gidx[pl.ds(t * L, L)]
        mask = pos >= m0
        plsc.addupdate_scatter(acc, [lane_rows + (off + g)], ones, mask=mask)
      return carry

    lax.fori_loop(start, end, cbody, 0)

  do_edges(up0v, up0g, s_up0, bat0, 0 * NG, 100000)
  do_edges(bd1v, bd1g, s_bd1, bat0, 0 * NG, 100000)
  do_edges(d1v, d1g, s_d1, bat1, 1 * NG, 200000)
  do_edges(u1v, u1g, s_u1, bat1, 1 * NG, 90000)
  do_edges(bd2v, bd2g, s_bd2, bat1, 1 * NG, 45000)
  do_edges(d2v, d2g, s_d2, bat2, 2 * NG, 60000)
  do_counts(bat0, N0, 3 * NG)
  do_counts(bat1, N1, 4 * NG)
  do_counts(bat2, N2, 5 * NG)

  pltpu.sync_copy(acc, outr.at[wid])


_stage_b = functools.partial(
    pl.kernel,
    out_type=jax.ShapeDtypeStruct((NW, L * ROW), _f32),
    mesh=_MESH,
    compiler_params=_SC_PARAMS,
    scratch_types=[
        pltpu.VMEM((L * ROW,), _f32),
        pltpu.VMEM((CB,), _i32),
        pltpu.VMEM((CB,), _i32),
        pltpu.VMEM((CB,), _f32),
        pltpu.VMEM((CB,), _i32),
        pltpu.SemaphoreType.DMA,
    ],
)(_stage_b_body)


def _wsum_body(w_ref, o_ref):
  o_ref[...] = jnp.sum(w_ref[...], axis=2)


def _combine_body(t0, t1, t2, n0, n1, n2, b_ref, o_ref):
  T0 = jnp.sum(t0[...], axis=0)
  T1 = jnp.sum(t1[...], axis=0)
  T2 = jnp.sum(t2[...], axis=0)
  c0v = jnp.sum(n0[...], axis=0)
  c1v = jnp.sum(n1[...], axis=0)
  c2v = jnp.sum(n2[...], axis=0)
  b0s = jnp.sum(b_ref[0:2, :])
  b1s = jnp.sum(b_ref[2:4, :]) + jnp.sum(b_ref[5:6, :])
  b2s = jnp.sum(b_ref[4:5, :])
  tot = (T0 + c0v * b0s) / jnp.maximum(c0v, 1.0)
  tot = tot + (T1 + c1v * b1s) / jnp.maximum(c1v, 1.0)
  tot = tot + (T2 + c2v * b2s) / jnp.maximum(c2v, 1.0)
  o_ref[...] = jax.nn.sigmoid(tot)[None, :]


def kernel(x0, x1, x2, up_index_0, boundary_index_1, down_index_1, up_index_1,
           boundary_index_2, down_index_2, batch0, batch1, batch2,
           W0, b0, W1, b1, W2, b2, W3, b3, W4, b4, W5, b5):
  wstack = jnp.stack([W0, W1, W2, W3, W4, W5]).astype(_f32)
  bstack = jnp.stack([b0, b1, b2, b3, b4, b5]).astype(_f32)

  wsum = pl.pallas_call(
      _wsum_body,
      out_shape=jax.ShapeDtypeStruct((NSLOT, C), _f32),
  )(wstack)

  def r(a):
    return a.astype(_i32)

  sw = _stage_a(r(x0), r(x1), r(x2), wsum)

  partials = _stage_b(
      *sw,
      r(batch0), r(batch1), r(batch2),
      r(up_index_0[0]), r(up_index_0[1]),
      r(boundary_index_1[1]), r(boundary_index_1[0]),
      r(down_index_1[0]), r(down_index_1[1]),
      r(up_index_1[0]), r(up_index_1[1]),
      r(boundary_index_2[1]), r(boundary_index_2[0]),
      r(down_index_2[0]), r(down_index_2[1]),
      jnp.zeros((L * ROW,), _f32))

  p = partials.reshape(NW * L, NSLOT, NG)
  out = pl.pallas_call(
      _combine_body,
      out_shape=jax.ShapeDtypeStruct((1, NG), _f32),
  )(p[:, 0, :], p[:, 1, :], p[:, 2, :], p[:, 3, :], p[:, 4, :], p[:, 5, :],
    bstack)
  return out.reshape(NG)


# speedup vs baseline: 18.4152x; 1.1780x over previous
import functools
import math

import jax
import jax.numpy as jnp
from jax import lax
from jax.experimental import pallas as pl
from jax.experimental.pallas import tpu as pltpu
from jax.experimental.pallas import tpu_sc as plsc

C = 128
NG = 64
N0, N1, N2 = 50000, 50000, 15000
NSLOT = 6
ROW = NSLOT * NG
L = 16
NC, NS = 2, 16
NW = NC * NS
CB = 2048

_f32 = jnp.float32
_i32 = jnp.int32

_SC_PARAMS = pltpu.CompilerParams(needs_layout_passes=False)
_MESH = plsc.VectorSubcoreMesh(core_axis_name="c", subcore_axis_name="s",
                               num_cores=NC, num_subcores=NS)


def _wwin(total):
  return math.ceil(total / (NW * L)) * L


_WA = _wwin(N0)


def _wid():
  return lax.axis_index("s") * NC + lax.axis_index("c")


def _clamp8(raw, total, w):
  return pl.multiple_of(jnp.minimum(raw, total - w), 8)


def _stage_a_body(x0r, x1r, x2r, wsumr,
                  t_up0, t_bd1, t_d1, t_u1, t_bd2, t_d2,
                  wtab, xrows, bufa, bufb, bufc, sem):
  wid = _wid()
  lanes = lax.iota(_i32, L)
  pltpu.sync_copy(wsumr, wtab)

  def job_x(xr, N, tables):
    ww = _wwin(N)
    wbase = _clamp8(wid * ww, N, ww)
    pltpu.sync_copy(xr.at[pl.ds(wbase * 4, 4 * ww)],
                    xrows.at[pl.ds(0, 4 * ww)])

    for t in range(ww // L):
      pos = lanes + t * L
      f = pos * 4
      xv = [plsc.load_gather(xrows, [f + j]) for j in range(4)]
      for (_, w_slot, buf) in tables:
        wsl = jnp.full((L,), w_slot, _i32)
        val = plsc.load_gather(wtab, [wsl, xv[0]])
        for j in range(1, 4):
          val = val + plsc.load_gather(wtab, [wsl, xv[j]])
        buf[pl.ds(t * L, L)] = val
    for (out_ref, _, buf) in tables:
      pltpu.sync_copy(buf.at[pl.ds(0, ww)], out_ref.at[pl.ds(wbase, ww)])

  job_x(x0r, N0, [(t_up0, 0, bufa)])
  job_x(x1r, N1, [(t_bd1, 1, bufa), (t_d1, 2, bufb), (t_u1, 3, bufc)])
  job_x(x2r, N2, [(t_bd2, 5, bufa), (t_d2, 4, bufb)])


_stage_a = functools.partial(
    pl.kernel,
    out_type=(
        jax.ShapeDtypeStruct((N0,), _f32),
        jax.ShapeDtypeStruct((N1,), _f32),
        jax.ShapeDtypeStruct((N1,), _f32),
        jax.ShapeDtypeStruct((N1,), _f32),
        jax.ShapeDtypeStruct((N2,), _f32),
        jax.ShapeDtypeStruct((N2,), _f32),
    ),
    mesh=_MESH,
    compiler_params=_SC_PARAMS,
    scratch_types=[
        pltpu.VMEM((NSLOT, C), _f32),
        pltpu.VMEM((4 * _WA,), _i32),
        pltpu.VMEM((_WA,), _f32),
        pltpu.VMEM((_WA,), _f32),
        pltpu.VMEM((_WA,), _f32),
        pltpu.SemaphoreType.DMA,
    ],
)(_stage_a_body)


def _cpw(E):
  return math.ceil(math.ceil(E / CB) / NW)


def _stage_b_body(s_up0, s_bd1, s_d1, s_u1, s_bd2, s_d2,
                  rb0, rb1, rb2,
                  up0v, up0g, bd1v, bd1g, d1v, d1g, u1v, u1g,
                  bd2v, bd2g, d2v, d2g, zr, outr,
                  acc, vidx0, vidx1, gidx0, gidx1, swt, btab,
                  semLv, semLg, semSW, semBT):
  wid = _wid()
  lanes = lax.iota(_i32, L)
  lane_rows = lanes * ROW
  ones = jnp.ones((L,), _f32)
  vidxs = (vidx0, vidx1)
  gidxs = (gidx0, gidx1)

  pltpu.sync_copy(zr, acc)

  def chunk_m0_base(ci, E):
    m0 = jnp.maximum(0, ci * CB - (E - CB))
    return m0, pl.multiple_of(ci * CB - m0, 8)

  def do_edges(ev, eg, off, E):
    cpw = _cpw(E)
    start = wid * cpw
    m0s, Ls = {}, {}

    def issue_l(k):
      m0, base = chunk_m0_base(start + k, E)
      m0s[k] = m0
      b = k % 2
      Ls[k] = (
          pltpu.async_copy(ev.at[pl.ds(base, CB)], vidxs[b], semLv),
          pltpu.async_copy(eg.at[pl.ds(base, CB)], gidxs[b], semLg),
      )

    issue_l(0)
    for k in range(cpw):
      for cp in Ls[k]:
        cp.wait()
      if k + 1 < cpw:
        issue_l(k + 1)
      b = k % 2
      m0 = m0s[k]

      def tbody(t, carry):
        pos = lanes + t * L
        vi = plsc.load_gather(vidxs[b], [pos])
        gi = plsc.load_gather(gidxs[b], [pos])
        val = plsc.load_gather(swt, [vi])
        g = plsc.load_gather(btab, [gi])
        mask = pos >= m0
        plsc.addupdate_scatter(acc, [lane_rows + (off + g)], val, mask=mask)
        return carry

      lax.fori_loop(0, CB // L, tbody, 0)

  def do_counts(off, N):
    cpn = math.ceil(N / NW)
    a0 = wid * cpn
    a1 = jnp.minimum(N, a0 + cpn)

    def tbody(t, carry):
      idx = a0 + t * L + lanes
      mask = idx < a1
      g = plsc.load_gather(btab, [jnp.minimum(idx, N - 1)])
      plsc.addupdate_scatter(acc, [lane_rows + (off + g)], ones, mask=mask)
      return carry

    lax.fori_loop(0, math.ceil(cpn / L), tbody, 0)

  def load_sw(src, ln):
    return pltpu.async_copy(src, swt.at[pl.ds(0, ln)], semSW)

  def load_bat(src, ln):
    return pltpu.async_copy(src, btab.at[pl.ds(0, ln)], semBT)

  cpb = load_bat(rb0, N0)
  cps = load_sw(s_up0, N0)
  cpb.wait()
  cps.wait()
  do_edges(up0v, up0g, 0 * NG, 100000)
  load_sw(s_bd1, N1).wait()
  do_edges(bd1v, bd1g, 0 * NG, 100000)
  do_counts(3 * NG, N0)
  cpb = load_bat(rb1, N1)
  cps = load_sw(s_d1, N1)
  cpb.wait()
  cps.wait()
  do_edges(d1v, d1g, 1 * NG, 200000)
  load_sw(s_u1, N1).wait()
  do_edges(u1v, u1g, 1 * NG, 90000)
  load_sw(s_bd2, N2).wait()
  do_edges(bd2v, bd2g, 1 * NG, 45000)
  do_counts(4 * NG, N1)
  cpb = load_bat(rb2, N2)
  cps = load_sw(s_d2, N2)
  cpb.wait()
  cps.wait()
  do_edges(d2v, d2g, 2 * NG, 60000)
  do_counts(5 * NG, N2)

  pltpu.sync_copy(acc, outr.at[wid])


_stage_b = functools.partial(
    pl.kernel,
    out_type=jax.ShapeDtypeStruct((NW, L * ROW), _f32),
    mesh=_MESH,
    compiler_params=_SC_PARAMS,
    scratch_types=[
        pltpu.VMEM((L * ROW,), _f32),
        pltpu.VMEM((CB,), _i32),
        pltpu.VMEM((CB,), _i32),
        pltpu.VMEM((CB,), _i32),
        pltpu.VMEM((CB,), _i32),
        pltpu.VMEM((N0,), _f32),
        pltpu.VMEM((N0,), _i32),
        pltpu.SemaphoreType.DMA,
        pltpu.SemaphoreType.DMA,
        pltpu.SemaphoreType.DMA,
        pltpu.SemaphoreType.DMA,
    ],
)(_stage_b_body)


def _wsum_body(w_ref, o_ref):
  o_ref[...] = jnp.sum(w_ref[...], axis=2)


def _combine_body(t0, t1, t2, n0, n1, n2, b_ref, o_ref):
  T0 = jnp.sum(t0[...], axis=0)
  T1 = jnp.sum(t1[...], axis=0)
  T2 = jnp.sum(t2[...], axis=0)
  c0v = jnp.sum(n0[...], axis=0)
  c1v = jnp.sum(n1[...], axis=0)
  c2v = jnp.sum(n2[...], axis=0)
  b0s = jnp.sum(b_ref[0:2, :])
  b1s = jnp.sum(b_ref[2:4, :]) + jnp.sum(b_ref[5:6, :])
  b2s = jnp.sum(b_ref[4:5, :])
  tot = (T0 + c0v * b0s) / jnp.maximum(c0v, 1.0)
  tot = tot + (T1 + c1v * b1s) / jnp.maximum(c1v, 1.0)
  tot = tot + (T2 + c2v * b2s) / jnp.maximum(c2v, 1.0)
  o_ref[...] = jax.nn.sigmoid(tot)[None, :]


def kernel(x0, x1, x2, up_index_0, boundary_index_1, down_index_1, up_index_1,
           boundary_index_2, down_index_2, batch0, batch1, batch2,
           W0, b0, W1, b1, W2, b2, W3, b3, W4, b4, W5, b5):
  wstack = jnp.stack([W0, W1, W2, W3, W4, W5]).astype(_f32)
  bstack = jnp.stack([b0, b1, b2, b3, b4, b5]).astype(_f32)

  wsum = pl.pallas_call(
      _wsum_body,
      out_shape=jax.ShapeDtypeStruct((NSLOT, C), _f32),
  )(wstack)

  def r(a):
    return a.astype(_i32)

  sw = _stage_a(r(x0).reshape(-1), r(x1).reshape(-1), r(x2).reshape(-1),
                wsum)

  partials = _stage_b(
      *sw,
      r(batch0), r(batch1), r(batch2),
      r(up_index_0[0]), r(up_index_0[1]),
      r(boundary_index_1[1]), r(boundary_index_1[0]),
      r(down_index_1[0]), r(down_index_1[1]),
      r(up_index_1[0]), r(up_index_1[1]),
      r(boundary_index_2[1]), r(boundary_index_2[0]),
      r(down_index_2[0]), r(down_index_2[1]),
      jnp.zeros((L * ROW,), _f32))

  p = partials.reshape(NW * L, NSLOT, NG)
  out = pl.pallas_call(
      _combine_body,
      out_shape=jax.ShapeDtypeStruct((1, NG), _f32),
  )(p[:, 0, :], p[:, 1, :], p[:, 2, :], p[:, 3, :], p[:, 4, :], p[:, 5, :],
    bstack)
  return out.reshape(NG)
